# baseline (speedup 1.0000x reference)
import functools

import jax
import jax.numpy as jnp
from jax import lax
from jax.experimental import pallas as pl
from jax.experimental.pallas import tpu as pltpu, tpu_sc as plsc

_KEEP_RATE = 0.9

_NC, _NS, _L = 2, 16, 16
_NW = _NC * _NS


def _sc_gather(idx_flat, mask_flat, n_per_batch, n_graphs):
    tok = idx_flat.shape[0]
    tpw = tok // _NW
    mesh = plsc.VectorSubcoreMesh(
        core_axis_name="c", subcore_axis_name="s",
        num_cores=_NC, num_subcores=_NS)

    @functools.partial(
        pl.kernel,
        out_type=jax.ShapeDtypeStruct((tok,), jnp.float32),
        mesh=mesh,
        scratch_types=[
            pltpu.VMEM((tpw,), jnp.int32),
            pltpu.VMEM((mask_flat.shape[0],), jnp.float32),
            pltpu.VMEM((tpw,), jnp.float32),
        ],
        compiler_params=pltpu.CompilerParams(needs_layout_passes=False),
    )
    def k(idx_hbm, mask_hbm, out_hbm, idx_v, mask_v, out_v):
        wid = lax.axis_index("s") * _NC + lax.axis_index("c")
        base = wid * tpw
        table_off = (base // n_per_batch) * n_graphs
        pltpu.sync_copy(mask_hbm, mask_v)
        pltpu.sync_copy(idx_hbm.at[pl.ds(base, tpw)], idx_v)

        def body(i, carry):
            sl = pl.ds(i * _L, _L)
            out_v[sl] = plsc.load_gather(mask_v, [idx_v[sl] + table_off])
            return carry

        lax.fori_loop(0, tpw // _L, body, 0)
        pltpu.sync_copy(out_v, out_hbm.at[pl.ds(base, tpw)])

    return k(idx_flat, mask_flat)


def _tc_scale(x2, scale2):
    tok, d = x2.shape
    blk = 512
    inv_keep = 1.0 / _KEEP_RATE

    def body(x_ref, s_ref, o_ref):
        o_ref[...] = x_ref[...] * (s_ref[...] * inv_keep)

    return pl.pallas_call(
        body,
        grid=(tok // blk,),
        in_specs=[
            pl.BlockSpec((blk, d), lambda i: (i, 0)),
            pl.BlockSpec((blk, 1), lambda i: (i, 0)),
        ],
        out_specs=pl.BlockSpec((blk, d), lambda i: (i, 0)),
        out_shape=jax.ShapeDtypeStruct((tok, d), x2.dtype),
    )(x2, scale2)


def kernel(x, graph_idxs, graph_mask):
    b, n, d = x.shape
    n_graphs = graph_mask.shape[1]
    tok = b * n
    idx_flat = graph_idxs.astype(jnp.int32).reshape(tok)
    mask_flat = graph_mask.astype(jnp.float32).reshape(b * n_graphs)
    scale = _sc_gather(idx_flat, mask_flat, n, n_graphs)
    out2 = _tc_scale(x.reshape(tok, d), scale.reshape(tok, 1))
    return out2.reshape(b, n, d)


# speedup vs baseline: 1.1963x; 1.1963x over previous
import functools

import jax
import jax.numpy as jnp
from jax import lax
from jax.experimental import pallas as pl
from jax.experimental.pallas import tpu as pltpu, tpu_sc as plsc

_KEEP_RATE = 0.9

_NC, _NS, _L = 2, 16, 16
_NW = _NC * _NS


def _sc_gather(idx_flat, mask_flat, n_per_batch, n_graphs):
    tok = idx_flat.shape[0]
    tpw = tok // _NW
    mesh = plsc.VectorSubcoreMesh(
        core_axis_name="c", subcore_axis_name="s",
        num_cores=_NC, num_subcores=_NS)

    @functools.partial(
        pl.kernel,
        out_type=jax.ShapeDtypeStruct((tok,), jnp.float32),
        mesh=mesh,
        scratch_types=[
            pltpu.VMEM((tpw,), jnp.int32),
            pltpu.VMEM((mask_flat.shape[0],), jnp.float32),
            pltpu.VMEM((tpw,), jnp.float32),
        ],
        compiler_params=pltpu.CompilerParams(needs_layout_passes=False),
    )
    def k(idx_hbm, mask_hbm, out_hbm, idx_v, mask_v, out_v):
        wid = lax.axis_index("s") * _NC + lax.axis_index("c")
        base = wid * tpw
        table_off = (base // n_per_batch) * n_graphs
        pltpu.sync_copy(mask_hbm, mask_v)
        pltpu.sync_copy(idx_hbm.at[pl.ds(base, tpw)], idx_v)

        def body(i, carry):
            sl = pl.ds(i * _L, _L)
            out_v[sl] = plsc.load_gather(mask_v, [idx_v[sl] + table_off])
            return carry

        lax.fori_loop(0, tpw // _L, body, 0)
        pltpu.sync_copy(out_v, out_hbm.at[pl.ds(base, tpw)])

    return k(idx_flat, mask_flat)


def _tc_scale(x2, scale2):
    tok, d = x2.shape
    blk = 2048
    inv_keep = 1.0 / _KEEP_RATE

    def body(x_ref, s_ref, o_ref):
        o_ref[...] = x_ref[...] * (s_ref[...] * inv_keep)

    return pl.pallas_call(
        body,
        grid=(tok // blk,),
        in_specs=[
            pl.BlockSpec((blk, d), lambda i: (i, 0)),
            pl.BlockSpec((blk, 1), lambda i: (i, 0)),
        ],
        out_specs=pl.BlockSpec((blk, d), lambda i: (i, 0)),
        out_shape=jax.ShapeDtypeStruct((tok, d), x2.dtype),
    )(x2, scale2)


def kernel(x, graph_idxs, graph_mask):
    b, n, d = x.shape
    n_graphs = graph_mask.shape[1]
    tok = b * n
    idx_flat = graph_idxs.astype(jnp.int32).reshape(tok)
    mask_flat = graph_mask.astype(jnp.float32).reshape(b * n_graphs)
    scale = _sc_gather(idx_flat, mask_flat, n, n_graphs)
    out2 = _tc_scale(x.reshape(tok, d), scale.reshape(tok, 1))
    return out2.reshape(b, n, d)


# speedup vs baseline: 1.1980x; 1.0015x over previous
import functools

import jax
import jax.numpy as jnp
from jax import lax
from jax.experimental import pallas as pl
from jax.experimental.pallas import tpu as pltpu, tpu_sc as plsc

_KEEP_RATE = 0.9

_NC, _NS, _L = 2, 16, 16
_NW = _NC * _NS


def _sc_gather(idx_flat, mask_flat, n_per_batch, n_graphs):
    tok = idx_flat.shape[0]
    tpw = tok // _NW
    mesh = plsc.VectorSubcoreMesh(
        core_axis_name="c", subcore_axis_name="s",
        num_cores=_NC, num_subcores=_NS)

    @functools.partial(
        pl.kernel,
        out_type=jax.ShapeDtypeStruct((tok,), jnp.float32),
        mesh=mesh,
        scratch_types=[
            pltpu.VMEM((tpw,), jnp.int32),
            pltpu.VMEM((mask_flat.shape[0],), jnp.float32),
            pltpu.VMEM((tpw,), jnp.float32),
        ],
        compiler_params=pltpu.CompilerParams(needs_layout_passes=False),
    )
    def k(idx_hbm, mask_hbm, out_hbm, idx_v, mask_v, out_v):
        wid = lax.axis_index("s") * _NC + lax.axis_index("c")
        base = wid * tpw
        table_off = (base // n_per_batch) * n_graphs
        pltpu.sync_copy(mask_hbm, mask_v)
        pltpu.sync_copy(idx_hbm.at[pl.ds(base, tpw)], idx_v)

        def body(i, carry):
            sl = pl.ds(i * _L, _L)
            out_v[sl] = plsc.load_gather(mask_v, [idx_v[sl] + table_off])
            return carry

        lax.fori_loop(0, tpw // _L, body, 0)
        pltpu.sync_copy(out_v, out_hbm.at[pl.ds(base, tpw)])

    return k(idx_flat, mask_flat)


def _tc_scale(x2, scale2):
    tok, d = x2.shape
    blk = 4096
    inv_keep = 1.0 / _KEEP_RATE

    def body(x_ref, s_ref, o_ref):
        o_ref[...] = x_ref[...] * (s_ref[...] * inv_keep)

    return pl.pallas_call(
        body,
        grid=(tok // blk,),
        in_specs=[
            pl.BlockSpec((blk, d), lambda i: (i, 0)),
            pl.BlockSpec((blk, 1), lambda i: (i, 0)),
        ],
        out_specs=pl.BlockSpec((blk, d), lambda i: (i, 0)),
        out_shape=jax.ShapeDtypeStruct((tok, d), x2.dtype),
    )(x2, scale2)


def kernel(x, graph_idxs, graph_mask):
    b, n, d = x.shape
    n_graphs = graph_mask.shape[1]
    tok = b * n
    idx_flat = graph_idxs.astype(jnp.int32).reshape(tok)
    mask_flat = graph_mask.astype(jnp.float32).reshape(b * n_graphs)
    scale = _sc_gather(idx_flat, mask_flat, n, n_graphs)
    out2 = _tc_scale(x.reshape(tok, d), scale.reshape(tok, 1))
    return out2.reshape(b, n, d)


# speedup vs baseline: 1.2035x; 1.0045x over previous
import functools

import jax
import jax.numpy as jnp
from jax import lax
from jax.experimental import pallas as pl
from jax.experimental.pallas import tpu as pltpu, tpu_sc as plsc

_KEEP_RATE = 0.9

_NC, _NS, _L = 2, 16, 16
_NW = _NC * _NS


def _sc_gather(idx_flat, mask_flat, n_per_batch, n_graphs, rep):
    tok = idx_flat.shape[0]
    tpw = tok // _NW
    mesh = plsc.VectorSubcoreMesh(
        core_axis_name="c", subcore_axis_name="s",
        num_cores=_NC, num_subcores=_NS)

    @functools.partial(
        pl.kernel,
        out_type=jax.ShapeDtypeStruct((tok, rep), jnp.float32),
        mesh=mesh,
        scratch_types=[
            pltpu.VMEM((tpw,), jnp.int32),
            pltpu.VMEM((mask_flat.shape[0],), jnp.float32),
            pltpu.VMEM((_L,), jnp.float32),
            pltpu.VMEM((tpw, rep), jnp.float32),
        ],
        compiler_params=pltpu.CompilerParams(needs_layout_passes=False),
    )
    def k(idx_hbm, mask_hbm, out_hbm, idx_v, mask_v, g_v, out_v):
        wid = lax.axis_index("s") * _NC + lax.axis_index("c")
        base = wid * tpw
        table_off = (base // n_per_batch) * n_graphs
        pltpu.sync_copy(mask_hbm, mask_v)
        pltpu.sync_copy(idx_hbm.at[pl.ds(base, tpw)], idx_v)

        def body(i, carry):
            sl = pl.ds(i * _L, _L)
            g = plsc.load_gather(mask_v, [idx_v[sl] + table_off])
            for j in range(_L):
                splat = jnp.full((_L,), g[j], jnp.float32)
                for c in range(rep // _L):
                    out_v[i * _L + j, pl.ds(c * _L, _L)] = splat
            return carry

        lax.fori_loop(0, tpw // _L, body, 0)
        pltpu.sync_copy(out_v, out_hbm.at[pl.ds(base, tpw)])

    return k(idx_flat, mask_flat)


def _tc_scale(x2, scale2):
    tok, d = x2.shape
    rep = scale2.shape[1]
    blk = 2048
    inv_keep = 1.0 / _KEEP_RATE

    def body(x_ref, s_ref, o_ref):
        s = s_ref[...] * inv_keep
        sb = jnp.concatenate([s] * (d // rep), axis=1)
        o_ref[...] = x_ref[...] * sb

    return pl.pallas_call(
        body,
        grid=(tok // blk,),
        in_specs=[
            pl.BlockSpec((blk, d), lambda i: (i, 0)),
            pl.BlockSpec((blk, rep), lambda i: (i, 0)),
        ],
        out_specs=pl.BlockSpec((blk, d), lambda i: (i, 0)),
        out_shape=jax.ShapeDtypeStruct((tok, d), x2.dtype),
    )(x2, scale2)


def kernel(x, graph_idxs, graph_mask):
    b, n, d = x.shape
    n_graphs = graph_mask.shape[1]
    tok = b * n
    idx_flat = graph_idxs.astype(jnp.int32).reshape(tok)
    mask_flat = graph_mask.astype(jnp.float32).reshape(b * n_graphs)
    scale_rep = _sc_gather(idx_flat, mask_flat, n, n_graphs, rep=128)
    out2 = _tc_scale(x.reshape(tok, d), scale_rep)
    return out2.reshape(b, n, d)


# speedup vs baseline: 1.4048x; 1.1673x over previous
import functools

import jax
import jax.numpy as jnp
from jax import lax
from jax.experimental import pallas as pl
from jax.experimental.pallas import tpu as pltpu, tpu_sc as plsc

_KEEP_RATE = 0.9

_NC, _NS, _L = 2, 16, 16
_NW = _NC * _NS


def _sc_gather(idx_flat, mask_flat, n_per_batch, n_graphs):
    tok = idx_flat.shape[0]
    tpw = tok // _NW
    mesh = plsc.VectorSubcoreMesh(
        core_axis_name="c", subcore_axis_name="s",
        num_cores=_NC, num_subcores=_NS)

    @functools.partial(
        pl.kernel,
        out_type=jax.ShapeDtypeStruct((tok,), jnp.float32),
        mesh=mesh,
        scratch_types=[
            pltpu.VMEM((tpw,), jnp.int32),
            pltpu.VMEM((mask_flat.shape[0],), jnp.float32),
            pltpu.VMEM((tpw,), jnp.float32),
        ],
        compiler_params=pltpu.CompilerParams(needs_layout_passes=False),
    )
    def k(idx_hbm, mask_hbm, out_hbm, idx_v, mask_v, out_v):
        wid = lax.axis_index("s") * _NC + lax.axis_index("c")
        base = wid * tpw
        table_off = (base // n_per_batch) * n_graphs
        pltpu.sync_copy(mask_hbm, mask_v)
        pltpu.sync_copy(idx_hbm.at[pl.ds(base, tpw)], idx_v)

        def body(i, carry):
            sl = pl.ds(i * _L, _L)
            out_v[sl] = plsc.load_gather(mask_v, [idx_v[sl] + table_off])
            return carry

        lax.fori_loop(0, tpw // _L, body, 0)
        pltpu.sync_copy(out_v, out_hbm.at[pl.ds(base, tpw)])

    return k(idx_flat, mask_flat)


def _tc_scale(x2, scale2):
    tok, d = x2.shape
    blk = 2048
    rows = blk // 128
    inv_keep = 1.0 / _KEEP_RATE

    def body(x_ref, s_ref, o_ref):
        s_t = jnp.transpose(s_ref[...] * inv_keep)
        for r in range(rows):
            sl = pl.ds(r * 128, 128)
            o_ref[sl, :] = x_ref[sl, :] * s_t[:, r:r + 1]

    return pl.pallas_call(
        body,
        grid=(tok // blk,),
        in_specs=[
            pl.BlockSpec((blk, d), lambda i: (i, 0)),
            pl.BlockSpec((rows, 128), lambda i: (i, 0)),
        ],
        out_specs=pl.BlockSpec((blk, d), lambda i: (i, 0)),
        out_shape=jax.ShapeDtypeStruct((tok, d), x2.dtype),
    )(x2, scale2)


def kernel(x, graph_idxs, graph_mask):
    b, n, d = x.shape
    n_graphs = graph_mask.shape[1]
    tok = b * n
    idx_flat = graph_idxs.astype(jnp.int32).reshape(tok)
    mask_flat = graph_mask.astype(jnp.float32).reshape(b * n_graphs)
    scale = _sc_gather(idx_flat, mask_flat, n, n_graphs)
    out2 = _tc_scale(x.reshape(tok, d), scale.reshape(tok // 128, 128))
    return out2.reshape(b, n, d)


# speedup vs baseline: 1.4469x; 1.0300x over previous
import functools

import jax
import jax.numpy as jnp
from jax import lax
from jax.experimental import pallas as pl
from jax.experimental.pallas import tpu as pltpu, tpu_sc as plsc

_KEEP_RATE = 0.9

_NC, _NS, _L = 2, 16, 16
_NW = _NC * _NS


def _sc_gather(idx, mask_flat):
    b, n = idx.shape
    n_graphs = mask_flat.shape[0] // b
    tok = b * n
    tpw = tok // _NW
    wpr = n // tpw
    mesh = plsc.VectorSubcoreMesh(
        core_axis_name="c", subcore_axis_name="s",
        num_cores=_NC, num_subcores=_NS)

    @functools.partial(
        pl.kernel,
        out_type=jax.ShapeDtypeStruct((tok,), jnp.float32),
        mesh=mesh,
        scratch_types=[
            pltpu.VMEM((tpw,), jnp.int32),
            pltpu.VMEM((mask_flat.shape[0],), jnp.float32),
            pltpu.VMEM((tpw,), jnp.float32),
        ],
        compiler_params=pltpu.CompilerParams(needs_layout_passes=False),
    )
    def k(idx_hbm, mask_hbm, out_hbm, idx_v, mask_v, out_v):
        wid = lax.axis_index("s") * _NC + lax.axis_index("c")
        row = wid // wpr
        col = (wid % wpr) * tpw
        table_off = row * n_graphs
        pltpu.sync_copy(mask_hbm, mask_v)
        pltpu.sync_copy(idx_hbm.at[row, pl.ds(col, tpw)], idx_v)

        def body(i, carry):
            sl = pl.ds(i * _L, _L)
            out_v[sl] = plsc.load_gather(mask_v, [idx_v[sl] + table_off])
            return carry

        lax.fori_loop(0, tpw // _L, body, 0)
        pltpu.sync_copy(out_v, out_hbm.at[pl.ds(row * n + col, tpw)])

    return k(idx, mask_flat)


def _tc_scale(x2, scale2):
    tok, d = x2.shape
    blk = 2048
    rows = blk // 128
    inv_keep = 1.0 / _KEEP_RATE

    def body(x_ref, s_ref, o_ref):
        s_t = jnp.transpose(s_ref[...] * inv_keep)
        for r in range(rows):
            sl = pl.ds(r * 128, 128)
            o_ref[sl, :] = x_ref[sl, :] * s_t[:, r:r + 1]

    return pl.pallas_call(
        body,
        grid=(tok // blk,),
        in_specs=[
            pl.BlockSpec((blk, d), lambda i: (i, 0)),
            pl.BlockSpec((rows, 128), lambda i: (i, 0)),
        ],
        out_specs=pl.BlockSpec((blk, d), lambda i: (i, 0)),
        out_shape=jax.ShapeDtypeStruct((tok, d), x2.dtype),
    )(x2, scale2)


def kernel(x, graph_idxs, graph_mask):
    b, n, d = x.shape
    n_graphs = graph_mask.shape[1]
    tok = b * n
    mask_flat = graph_mask.astype(jnp.float32).reshape(b * n_graphs)
    scale = _sc_gather(graph_idxs.astype(jnp.int32), mask_flat)
    out2 = _tc_scale(x.reshape(tok, d), scale.reshape(tok // 128, 128))
    return out2.reshape(b, n, d)
